# baseline (device time: 65301 ns/iter reference)
import jax
import jax.numpy as jnp
from jax import lax
from jax.experimental import pallas as pl
from jax.experimental.pallas import tpu as pltpu

N_DEV = 8
B = 2
S_LOC = 128
S = N_DEV * S_LOC
D = 512
H_LOC = 8
DH = 64
SCALE = 0.125


def kernel(x, Wq, Wo, Wk, Wv):
    def body(x_ref, wq_ref, wo_ref, wk_ref, wv_ref, out_ref,
             xg_ref, q_ref, k_ref, v_ref, ss_ref, rs_ref,
             ag_send, ag_recv, rs_send, rs_recv):
        my = lax.axis_index("i")

        barrier = pltpu.get_barrier_semaphore()
        for j in range(1, N_DEV):
            pl.semaphore_signal(barrier, inc=1,
                                device_id=(lax.rem(my + j, N_DEV),),
                                device_id_type=pl.DeviceIdType.MESH)
        pl.semaphore_wait(barrier, N_DEV - 1)

        xg_ref[pl.ds(my, 1)] = x_ref[...].astype(jnp.bfloat16)[None]
        ag = []
        for j in range(1, N_DEV):
            tgt = lax.rem(my + j, N_DEV)
            rdma = pltpu.make_async_remote_copy(
                src_ref=xg_ref.at[my],
                dst_ref=xg_ref.at[my],
                send_sem=ag_send.at[j - 1],
                recv_sem=ag_recv.at[j - 1],
                device_id=(tgt,),
                device_id_type=pl.DeviceIdType.MESH,
            )
            rdma.start()
            ag.append(rdma)

        wq = (wq_ref[...] * SCALE).astype(jnp.bfloat16)
        wk = wk_ref[...].astype(jnp.bfloat16)
        wv = wv_ref[...].astype(jnp.bfloat16)
        wo = wo_ref[...].astype(jnp.bfloat16)
        w_qkv = jnp.concatenate([wq, wk, wv], axis=1)

        def qkv_chunk(c):
            xc = xg_ref[pl.ds(c, 1)][0].reshape(B * S_LOC, D)
            qkv = jnp.dot(xc, w_qkv, preferred_element_type=jnp.float32)
            qkv = qkv.astype(jnp.bfloat16)
            off = pl.ds(c * S_LOC, S_LOC)
            q_ref[:, off] = qkv[:, :D].reshape(B, S_LOC, D)
            k_ref[:, off] = qkv[:, D:2 * D].reshape(B, S_LOC, D)
            v_ref[:, off] = qkv[:, 2 * D:].reshape(B, S_LOC, D)

        qkv_chunk(my)
        for j in range(1, N_DEV):
            ag[j - 1].wait_recv()
            qkv_chunk(lax.rem(my - j + N_DEV, N_DEV))

        def y_chunk(c):
            ys = []
            for b in range(B):
                qc = q_ref[b, pl.ds(c * S_LOC, S_LOC)]
                kb = k_ref[b]
                vb = v_ref[b]
                cols = []
                for h in range(H_LOC):
                    sl = slice(h * DH, (h + 1) * DH)
                    s = lax.dot_general(
                        qc[:, sl], kb[:, sl], (((1,), (1,)), ((), ())),
                        preferred_element_type=jnp.float32,
                    )
                    p = jnp.exp(s.astype(jnp.bfloat16))
                    l = jnp.sum(p, axis=1, keepdims=True,
                                dtype=jnp.float32)
                    o = jnp.dot(p, vb[:, sl],
                                preferred_element_type=jnp.float32)
                    cols.append((o / l).astype(jnp.bfloat16))
                att = jnp.concatenate(cols, axis=1)
                ys.append(jnp.dot(att, wo,
                                  preferred_element_type=jnp.float32))
            return jnp.stack(ys, axis=0).astype(jnp.bfloat16)

        rs = []
        for j in range(1, N_DEV):
            tgt = lax.rem(my + j, N_DEV)
            ss_ref[j - 1] = y_chunk(tgt)
            rdma = pltpu.make_async_remote_copy(
                src_ref=ss_ref.at[j - 1],
                dst_ref=rs_ref.at[j - 1],
                send_sem=rs_send.at[j - 1],
                recv_sem=rs_recv.at[j - 1],
                device_id=(tgt,),
                device_id_type=pl.DeviceIdType.MESH,
            )
            rdma.start()
            rs.append(rdma)

        acc = y_chunk(my).astype(jnp.float32)
        for j in range(1, N_DEV):
            rs[j - 1].wait_recv()
            acc = acc + rs_ref[j - 1].astype(jnp.float32)
        out_ref[...] = acc

        for r in ag + rs:
            r.wait_send()

        def _exit(second_barrier):
            for j in range(1, N_DEV):
                pl.semaphore_signal(second_barrier, inc=1,
                                    device_id=(lax.rem(my + j, N_DEV),),
                                    device_id_type=pl.DeviceIdType.MESH)
            pl.semaphore_wait(second_barrier, N_DEV - 1)

        pl.run_scoped(_exit, second_barrier=pltpu.SemaphoreType.REGULAR)

    return pl.pallas_call(
        body,
        out_shape=jax.ShapeDtypeStruct((B, S_LOC, D), jnp.float32),
        in_specs=[pl.BlockSpec(memory_space=pltpu.VMEM)] * 5,
        out_specs=pl.BlockSpec(memory_space=pltpu.VMEM),
        scratch_shapes=[
            pltpu.VMEM((N_DEV, B, S_LOC, D), jnp.bfloat16),
            pltpu.VMEM((B, S, D), jnp.bfloat16),
            pltpu.VMEM((B, S, D), jnp.bfloat16),
            pltpu.VMEM((B, S, D), jnp.bfloat16),
            pltpu.VMEM((N_DEV - 1, B, S_LOC, D), jnp.bfloat16),
            pltpu.VMEM((N_DEV - 1, B, S_LOC, D), jnp.bfloat16),
            pltpu.SemaphoreType.DMA((N_DEV - 1,)),
            pltpu.SemaphoreType.DMA((N_DEV - 1,)),
            pltpu.SemaphoreType.DMA((N_DEV - 1,)),
            pltpu.SemaphoreType.DMA((N_DEV - 1,)),
        ],
        compiler_params=pltpu.CompilerParams(
            collective_id=0, vmem_limit_bytes=96 * 1024 * 1024,
        ),
    )(x, Wq, Wo, Wk, Wv)


# device time: 59783 ns/iter; 1.0923x vs baseline; 1.0923x over previous
import jax
import jax.numpy as jnp
from jax import lax
from jax.experimental import pallas as pl
from jax.experimental.pallas import tpu as pltpu

N_DEV = 8
B = 2
S_LOC = 128
S = N_DEV * S_LOC
D = 512
H_LOC = 8
DH = 64
SCALE = 0.125


def kernel(x, Wq, Wo, Wk, Wv):
    def body(x_ref, wq_ref, wo_ref, wk_ref, wv_ref, out_ref,
             xg_ref, q_ref, k_ref, v_ref, ss_ref, rs_ref,
             ag_send, ag_recv, rs_send, rs_recv):
        my = lax.axis_index("i")

        barrier = pltpu.get_barrier_semaphore()
        for j in range(1, N_DEV):
            pl.semaphore_signal(barrier, inc=1,
                                device_id=(lax.rem(my + j, N_DEV),),
                                device_id_type=pl.DeviceIdType.MESH)
        pl.semaphore_wait(barrier, N_DEV - 1)

        xg_ref[pl.ds(my, 1)] = x_ref[...].astype(jnp.bfloat16)[None]
        ag = []
        for j in range(1, N_DEV):
            tgt = lax.rem(my + j, N_DEV)
            rdma = pltpu.make_async_remote_copy(
                src_ref=xg_ref.at[my],
                dst_ref=xg_ref.at[my],
                send_sem=ag_send.at[j - 1],
                recv_sem=ag_recv.at[j - 1],
                device_id=(tgt,),
                device_id_type=pl.DeviceIdType.MESH,
            )
            rdma.start()
            ag.append(rdma)

        wq = (wq_ref[...] * SCALE).astype(jnp.bfloat16)
        wk = wk_ref[...].astype(jnp.bfloat16)
        wv = wv_ref[...].astype(jnp.bfloat16)
        wo = wo_ref[...].astype(jnp.bfloat16)
        w_qkv = jnp.concatenate([wq, wk, wv], axis=1)

        def qkv_chunk(c):
            xc = xg_ref[pl.ds(c, 1)][0].reshape(B * S_LOC, D)
            qkv = jnp.dot(xc, w_qkv, preferred_element_type=jnp.float32)
            qkv = qkv.astype(jnp.bfloat16)
            rot = lax.rem(c - my - 1 + 2 * N_DEV, N_DEV)
            q_ref[:, pl.ds(rot * S_LOC, S_LOC)] = qkv[:, :D].reshape(B, S_LOC, D)
            off = pl.ds(c * S_LOC, S_LOC)
            k_ref[:, off] = qkv[:, D:2 * D].reshape(B, S_LOC, D)
            v_ref[:, off] = qkv[:, 2 * D:].reshape(B, S_LOC, D)

        qkv_chunk(my)
        for j in range(1, N_DEV):
            ag[j - 1].wait_recv()
            qkv_chunk(lax.rem(my - j + N_DEV, N_DEV))

        N_BLK = 2
        ROWS = S // N_BLK

        def y_block(t):
            ys = []
            for b in range(B):
                qb = q_ref[b, t * ROWS:(t + 1) * ROWS]
                kb = k_ref[b]
                vb = v_ref[b]
                cols = []
                for h in range(H_LOC):
                    sl = slice(h * DH, (h + 1) * DH)
                    s = lax.dot_general(
                        qb[:, sl], kb[:, sl], (((1,), (1,)), ((), ())),
                        preferred_element_type=jnp.float32,
                    )
                    p = jnp.exp(s.astype(jnp.bfloat16))
                    l = jnp.sum(p, axis=1, keepdims=True,
                                dtype=jnp.float32)
                    o = jnp.dot(p, vb[:, sl],
                                preferred_element_type=jnp.float32)
                    cols.append((o / l).astype(jnp.bfloat16))
                att = jnp.concatenate(cols, axis=1)
                ys.append(jnp.dot(att, wo,
                                  preferred_element_type=jnp.float32))
            return jnp.stack(ys, axis=0).astype(jnp.bfloat16)

        rs = []
        acc = None
        for t in range(N_BLK):
            y_blk = y_block(t)
            for i in range(ROWS // S_LOC):
                j = t * (ROWS // S_LOC) + i + 1
                rows = slice(i * S_LOC, (i + 1) * S_LOC)
                if j == N_DEV:
                    acc = y_blk[:, rows].astype(jnp.float32)
                    break
                tgt = lax.rem(my + j, N_DEV)
                ss_ref[j - 1] = y_blk[:, rows]
                rdma = pltpu.make_async_remote_copy(
                    src_ref=ss_ref.at[j - 1],
                    dst_ref=rs_ref.at[j - 1],
                    send_sem=rs_send.at[j - 1],
                    recv_sem=rs_recv.at[j - 1],
                    device_id=(tgt,),
                    device_id_type=pl.DeviceIdType.MESH,
                )
                rdma.start()
                rs.append(rdma)
        for j in range(1, N_DEV):
            rs[j - 1].wait_recv()
            acc = acc + rs_ref[j - 1].astype(jnp.float32)
        out_ref[...] = acc

        for r in ag + rs:
            r.wait_send()

        def _exit(second_barrier):
            for j in range(1, N_DEV):
                pl.semaphore_signal(second_barrier, inc=1,
                                    device_id=(lax.rem(my + j, N_DEV),),
                                    device_id_type=pl.DeviceIdType.MESH)
            pl.semaphore_wait(second_barrier, N_DEV - 1)

        pl.run_scoped(_exit, second_barrier=pltpu.SemaphoreType.REGULAR)

    return pl.pallas_call(
        body,
        out_shape=jax.ShapeDtypeStruct((B, S_LOC, D), jnp.float32),
        in_specs=[pl.BlockSpec(memory_space=pltpu.VMEM)] * 5,
        out_specs=pl.BlockSpec(memory_space=pltpu.VMEM),
        scratch_shapes=[
            pltpu.VMEM((N_DEV, B, S_LOC, D), jnp.bfloat16),
            pltpu.VMEM((B, S, D), jnp.bfloat16),
            pltpu.VMEM((B, S, D), jnp.bfloat16),
            pltpu.VMEM((B, S, D), jnp.bfloat16),
            pltpu.VMEM((N_DEV - 1, B, S_LOC, D), jnp.bfloat16),
            pltpu.VMEM((N_DEV - 1, B, S_LOC, D), jnp.bfloat16),
            pltpu.SemaphoreType.DMA((N_DEV - 1,)),
            pltpu.SemaphoreType.DMA((N_DEV - 1,)),
            pltpu.SemaphoreType.DMA((N_DEV - 1,)),
            pltpu.SemaphoreType.DMA((N_DEV - 1,)),
        ],
        compiler_params=pltpu.CompilerParams(
            collective_id=0, vmem_limit_bytes=96 * 1024 * 1024,
        ),
    )(x, Wq, Wo, Wk, Wv)


# device time: 54937 ns/iter; 1.1887x vs baseline; 1.0882x over previous
import jax
import jax.numpy as jnp
from jax import lax
from jax.experimental import pallas as pl
from jax.experimental.pallas import tpu as pltpu

N_DEV = 8
B = 2
S_LOC = 128
S = N_DEV * S_LOC
D = 512
H_LOC = 8
DH = 64
SCALE = 0.125


def kernel(x, Wq, Wo, Wk, Wv):
    def body(x_ref, wq_ref, wo_ref, wk_ref, wv_ref, out_ref,
             xg_ref, q_ref, k_ref, v_ref, ss_ref, rs_ref,
             ag_send, ag_recv, rs_send, rs_recv):
        my = lax.axis_index("i")

        barrier = pltpu.get_barrier_semaphore()
        for j in range(1, N_DEV):
            pl.semaphore_signal(barrier, inc=1,
                                device_id=(lax.rem(my + j, N_DEV),),
                                device_id_type=pl.DeviceIdType.MESH)
        pl.semaphore_wait(barrier, N_DEV - 1)

        xg_ref[pl.ds(my, 1)] = x_ref[...].astype(jnp.bfloat16)[None]
        ag = []
        for j in range(1, N_DEV):
            tgt = lax.rem(my + j, N_DEV)
            rdma = pltpu.make_async_remote_copy(
                src_ref=xg_ref.at[my],
                dst_ref=xg_ref.at[my],
                send_sem=ag_send.at[j - 1],
                recv_sem=ag_recv.at[j - 1],
                device_id=(tgt,),
                device_id_type=pl.DeviceIdType.MESH,
            )
            rdma.start()
            ag.append(rdma)

        wq = (wq_ref[...] * SCALE).astype(jnp.bfloat16)
        wk = wk_ref[...].astype(jnp.bfloat16)
        wv = wv_ref[...].astype(jnp.bfloat16)
        wo = wo_ref[...].astype(jnp.bfloat16)
        w_qkv = jnp.concatenate([wq, wk, wv], axis=1)

        def qkv_chunk(c):
            xc = xg_ref[pl.ds(c, 1)][0].reshape(B * S_LOC, D)
            qkv = jnp.dot(xc, w_qkv, preferred_element_type=jnp.float32)
            qkv = qkv.astype(jnp.bfloat16)
            rot = lax.rem(my - c + N_DEV, N_DEV)
            off = pl.ds(rot * S_LOC, S_LOC)
            q_ref[:, off] = qkv[:, :D].reshape(B, S_LOC, D)
            k_ref[:, off] = qkv[:, D:2 * D].reshape(B, S_LOC, D)
            v_ref[:, off] = qkv[:, 2 * D:].reshape(B, S_LOC, D)

        def attn_pass(q_lo, q_hi, k_lo, k_hi, prev=None):
            res = {}
            for b in range(B):
                qb = q_ref[b, q_lo:q_hi]
                kb = k_ref[b, k_lo:k_hi]
                vb = v_ref[b, k_lo:k_hi]
                for h in range(H_LOC):
                    sl = slice(h * DH, (h + 1) * DH)
                    s = lax.dot_general(
                        qb[:, sl], kb[:, sl], (((1,), (1,)), ((), ())),
                        preferred_element_type=jnp.float32,
                    )
                    p = jnp.exp(s.astype(jnp.bfloat16))
                    l = jnp.sum(p, axis=1, keepdims=True, dtype=jnp.float32)
                    o = jnp.dot(p, vb[:, sl],
                                preferred_element_type=jnp.float32)
                    if prev is not None:
                        po, plsum = prev[(b, h)]
                        o, l = po + o, plsum + l
                    res[(b, h)] = (o, l)
            return res

        def finalize(res):
            ys = []
            for b in range(B):
                cols = [(res[(b, h)][0] / res[(b, h)][1]).astype(jnp.bfloat16)
                        for h in range(H_LOC)]
                att = jnp.concatenate(cols, axis=1)
                ys.append(jnp.dot(att, wo,
                                  preferred_element_type=jnp.float32))
            return jnp.stack(ys, axis=0).astype(jnp.bfloat16)

        def send_y(y_blk, i, j):
            tgt = lax.rem(my - j + N_DEV, N_DEV)
            ss_ref[j - 1] = y_blk[:, i * S_LOC:(i + 1) * S_LOC]
            rdma = pltpu.make_async_remote_copy(
                src_ref=ss_ref.at[j - 1],
                dst_ref=rs_ref.at[j - 1],
                send_sem=rs_send.at[j - 1],
                recv_sem=rs_recv.at[j - 1],
                device_id=(tgt,),
                device_id_type=pl.DeviceIdType.MESH,
            )
            rdma.start()
            return rdma

        qkv_chunk(my)
        for j in range(1, 5):
            ag[j - 1].wait_recv()
            qkv_chunk(lax.rem(my - j + N_DEV, N_DEV))

        r0 = attn_pass(S_LOC, 5 * S_LOC, 0, 4 * S_LOC)

        for j in range(5, N_DEV):
            ag[j - 1].wait_recv()
            qkv_chunk(lax.rem(my - j + N_DEV, N_DEV))

        r0 = attn_pass(S_LOC, 5 * S_LOC, 4 * S_LOC, S, prev=r0)
        y0 = finalize(r0)
        rs = [send_y(y0, i, i + 1) for i in range(4)]

        y1 = finalize(attn_pass(5 * S_LOC, S, 0, S))
        rs += [send_y(y1, i, i + 5) for i in range(3)]

        acc = finalize(attn_pass(0, S_LOC, 0, S)).astype(jnp.float32)
        for j in range(1, N_DEV):
            rs[j - 1].wait_recv()
            acc = acc + rs_ref[j - 1].astype(jnp.float32)
        out_ref[...] = acc

        for r in ag + rs:
            r.wait_send()

        def _exit(second_barrier):
            for j in range(1, N_DEV):
                pl.semaphore_signal(second_barrier, inc=1,
                                    device_id=(lax.rem(my + j, N_DEV),),
                                    device_id_type=pl.DeviceIdType.MESH)
            pl.semaphore_wait(second_barrier, N_DEV - 1)

        pl.run_scoped(_exit, second_barrier=pltpu.SemaphoreType.REGULAR)

    return pl.pallas_call(
        body,
        out_shape=jax.ShapeDtypeStruct((B, S_LOC, D), jnp.float32),
        in_specs=[pl.BlockSpec(memory_space=pltpu.VMEM)] * 5,
        out_specs=pl.BlockSpec(memory_space=pltpu.VMEM),
        scratch_shapes=[
            pltpu.VMEM((N_DEV, B, S_LOC, D), jnp.bfloat16),
            pltpu.VMEM((B, S, D), jnp.bfloat16),
            pltpu.VMEM((B, S, D), jnp.bfloat16),
            pltpu.VMEM((B, S, D), jnp.bfloat16),
            pltpu.VMEM((N_DEV - 1, B, S_LOC, D), jnp.bfloat16),
            pltpu.VMEM((N_DEV - 1, B, S_LOC, D), jnp.bfloat16),
            pltpu.SemaphoreType.DMA((N_DEV - 1,)),
            pltpu.SemaphoreType.DMA((N_DEV - 1,)),
            pltpu.SemaphoreType.DMA((N_DEV - 1,)),
            pltpu.SemaphoreType.DMA((N_DEV - 1,)),
        ],
        compiler_params=pltpu.CompilerParams(
            collective_id=0, vmem_limit_bytes=96 * 1024 * 1024,
        ),
    )(x, Wq, Wo, Wk, Wv)
